# baseline (speedup 1.0000x reference)
import functools

import jax
import jax.numpy as jnp
from jax import lax
from jax.experimental import pallas as pl
from jax.experimental.pallas import tpu as pltpu
from jax.experimental.pallas import tpu_sc as plsc

_NC = 2
_NS = 16
_L = 16


def _phase1(coor_2d, np_pad, N, interpret=False):
    B = coor_2d.shape[0]
    P = coor_2d.shape[1] // 3
    mesh = plsc.VectorSubcoreMesh(
        core_axis_name="c", subcore_axis_name="s",
        num_cores=_NC, num_subcores=_NS)

    @functools.partial(
        pl.kernel,
        out_type=(
            jax.ShapeDtypeStruct((B, P), jnp.int32),
            jax.ShapeDtypeStruct((B, 16), jnp.int32),
        ),
        mesh=mesh,
        scratch_types=[
            pltpu.VMEM((P * 3,), jnp.float32),
            pltpu.VMEM((P,), jnp.int32),
            pltpu.VMEM((16,), jnp.int32),
            pltpu.VMEM((16,), jnp.int32),
        ],
        compiler_params=pltpu.CompilerParams(needs_layout_passes=False),
        interpret=interpret,
    )
    def k(coor_hbm, np_hbm, destp_hbm, cnts_hbm, coorb, destb, cntrow, npb):
        wid = lax.axis_index("s") * _NC + lax.axis_index("c")

        @pl.when(wid < B)
        def _():
            b = wid
            pltpu.sync_copy(coor_hbm.at[b], coorb)
            pltpu.sync_copy(np_hbm, npb)
            iota = lax.iota(jnp.int32, 16)
            zeros16 = jnp.zeros((16,), jnp.int32)
            npv = npb[...]

            def step(v, runs):
                pidx = v * 16 + iota
                camf = plsc.load_gather(coorb, [pidx * 3])
                cam = camf.astype(jnp.int32)
                valid = pidx < jnp.max(jnp.where(iota == b, npv, 0))
                dflat = jnp.full((16,), -1, jnp.int32)
                new_runs = []
                for n in range(N):
                    msk = (cam == n) & valid
                    inc = msk.astype(jnp.int32)
                    pos = plsc.cumsum(inc) + runs[n] - 1
                    dflat = jnp.where(msk, (b * N + n) * P + pos, dflat)
                    cnt = plsc.all_reduce_population_count(msk)
                    new_runs.append(runs[n] + cnt)
                destb[pl.ds(v * 16, 16)] = dflat
                return tuple(new_runs)

            init = tuple(jnp.zeros((16,), jnp.int32) for _ in range(N))
            runs = lax.fori_loop(0, P // 16, step, init)
            total = zeros16
            for n in range(N):
                total = jnp.where(iota == n, runs[n], total)
            cntrow[...] = total
            pltpu.sync_copy(destb, destp_hbm.at[b])
            pltpu.sync_copy(cntrow, cnts_hbm.at[b])

    return k(coor_2d, np_pad)


def _phase2a(destp, cnts, coor_2d, coor_2d_o, pts_all, pts_src, zrows,
             N, HW, W, interpret=False):
    B = coor_2d.shape[0]
    P = coor_2d.shape[1] // 3
    R = B * N
    CH = 1024
    SUB = 128
    NTILE = _NC * _NS
    per_tile = R * P // CH // NTILE
    parts = P // CH
    C = pts_src.shape[1]
    mesh = plsc.VectorSubcoreMesh(
        core_axis_name="c", subcore_axis_name="s",
        num_cores=_NC, num_subcores=_NS)

    @functools.partial(
        pl.kernel,
        out_type=(
            jax.ShapeDtypeStruct((R * P, C), jnp.float32),
            jax.ShapeDtypeStruct((R * 2 * P,), jnp.float32),
            jax.ShapeDtypeStruct((R * 2 * P,), jnp.float32),
            jax.ShapeDtypeStruct((3 * R * P,), jnp.float32),
            jax.ShapeDtypeStruct((R * P,), jnp.int32),
            jax.ShapeDtypeStruct((R * P,), jnp.int32),
        ),
        mesh=mesh,
        scratch_types=[
            pltpu.VMEM((P,), jnp.int32),
            pltpu.VMEM((P * 3,), jnp.float32),
            pltpu.VMEM((P * 3,), jnp.float32),
            pltpu.VMEM((P * 3,), jnp.float32),
            pltpu.VMEM((16,), jnp.int32),
            pltpu.VMEM((CH,), jnp.int32),
            pltpu.VMEM((CH,), jnp.int32),
            pltpu.VMEM((CH,), jnp.int32),
            pltpu.VMEM((SUB, C), jnp.float32),
            pltpu.VMEM((SUB, C), jnp.float32),
            pltpu.VMEM((SUB, C), jnp.float32),
            pltpu.VMEM((CH,), jnp.float32),
            pltpu.VMEM((CH,), jnp.float32),
            pltpu.VMEM((CH,), jnp.float32),
            pltpu.VMEM((CH,), jnp.float32),
            pltpu.VMEM((CH,), jnp.float32),
            pltpu.VMEM((CH,), jnp.float32),
            pltpu.VMEM((CH,), jnp.float32),
            pltpu.VMEM((CH,), jnp.int32),
            pltpu.SemaphoreType.DMA,
            pltpu.SemaphoreType.DMA,
        ],
        compiler_params=pltpu.CompilerParams(needs_layout_passes=False),
        interpret=interpret,
    )
    def k(destp_h, cnts_h, coor_h, cooro_h, pts_h, ptsrc_h, zrows_h,
          opf, oc, oco, op, om, oidxi,
          destb, coorb, coorob, ptsb, cntb, srcmap, idxp, idxi,
          rowbuf, rowbuf2, zerobuf, cxs, cys, oxs, oys, pxs, pys, pzs,
          mstage, sem, sem2):
        wid = lax.axis_index("s") * _NC + lax.axis_index("c")
        iota = lax.iota(jnp.int32, 16)
        fz = jnp.float32(0)
        bufs = (rowbuf, rowbuf2)
        sems = (sem, sem2)

        def drain(sx, k0, base, cnt, descs):
            gbase = base + sx * SUB
            nv = jnp.clip(cnt - (k0 + sx * SUB), 0, SUB)
            buf = bufs[sx % 2]

            @pl.when(nv > 0)
            def _():
                descs[sx].wait()

                def ztail(r2, _):
                    rsp = jnp.full((16,), r2, jnp.int32)
                    for c2 in range(C // 16):
                        plsc.store_scatter(
                            buf, [rsp, c2 * 16 + iota],
                            jnp.zeros((16,), jnp.float32))
                    return 0
                lax.fori_loop(nv, SUB, ztail, 0)
                pltpu.sync_copy(buf, opf.at[pl.ds(gbase, SUB)])

            @pl.when(nv == 0)
            def _():
                pltpu.sync_copy(zerobuf, opf.at[pl.ds(gbase, SUB)])

        pltpu.sync_copy(zrows_h, zerobuf)

        for j in range(per_tile):
            chunk = wid * per_tile + j
            row = chunk // parts
            part = chunk % parts
            b = row // N
            n = row % N
            k0 = part * CH
            base = row * P + k0

            pltpu.sync_copy(destp_h.at[b], destb)
            pltpu.sync_copy(coor_h.at[b], coorb)
            pltpu.sync_copy(cooro_h.at[b], coorob)
            pltpu.sync_copy(pts_h.at[b], ptsb)
            pltpu.sync_copy(cnts_h.at[b], cntb)

            def init_map(i, _):
                srcmap[pl.ds(i * 16, 16)] = jnp.full((16,), -1, jnp.int32)
                return 0
            lax.fori_loop(0, CH // 16, init_map, 0)

            def build_map(i, _):
                dvec = destb[pl.ds(i * 16, 16)]
                rel = dvec - base
                msk = (rel >= 0) & (rel < CH)
                plsc.store_scatter(srcmap, [rel], i * 16 + iota, mask=msk)
                return 0
            lax.fori_loop(0, P // 16, build_map, 0)

            cnt = jnp.max(jnp.where(iota == n, cntb[...], 0))

            def slots(i, _):
                pvec = srcmap[pl.ds(i * 16, 16)]
                vmsk = pvec >= 0
                psafe = jnp.where(vmsk, pvec, 0)
                p3 = psafe * 3
                cx = plsc.load_gather(coorb, [p3 + 1], mask=vmsk)
                cy = plsc.load_gather(coorb, [p3 + 2], mask=vmsk)
                ox = plsc.load_gather(coorob, [p3 + 1], mask=vmsk)
                oy = plsc.load_gather(coorob, [p3 + 2], mask=vmsk)
                px = plsc.load_gather(ptsb, [p3], mask=vmsk)
                py = plsc.load_gather(ptsb, [p3 + 1], mask=vmsk)
                pz = plsc.load_gather(ptsb, [p3 + 2], mask=vmsk)
                xi = (ox * 0.25).astype(jnp.int32)
                yi = (oy * 0.25).astype(jnp.int32)
                sl = pl.ds(i * 16, 16)
                cxs[sl] = jnp.where(vmsk, cx, fz)
                cys[sl] = jnp.where(vmsk, cy, fz)
                oxs[sl] = jnp.where(vmsk, xi.astype(jnp.float32), fz)
                oys[sl] = jnp.where(vmsk, yi.astype(jnp.float32), fz)
                pxs[sl] = jnp.where(vmsk, px, fz)
                pys[sl] = jnp.where(vmsk, py, fz)
                pzs[sl] = jnp.where(vmsk, pz, fz)
                mstage[sl] = psafe
                idxp[sl] = psafe + b * P
                idxi[sl] = jnp.where(vmsk, row * HW + yi * W + xi, 0)
                return 0
            lax.fori_loop(0, CH // 16, slots, 0)

            rk = row * 2 * P + k0
            pltpu.sync_copy(cxs, oc.at[pl.ds(rk, CH)])
            pltpu.sync_copy(cys, oc.at[pl.ds(rk + P, CH)])
            pltpu.sync_copy(oxs, oco.at[pl.ds(rk, CH)])
            pltpu.sync_copy(oys, oco.at[pl.ds(rk + P, CH)])
            rp = row * P + k0
            pltpu.sync_copy(pxs, op.at[pl.ds(rp, CH)])
            pltpu.sync_copy(pys, op.at[pl.ds(R * P + rp, CH)])
            pltpu.sync_copy(pzs, op.at[pl.ds(2 * R * P + rp, CH)])
            pltpu.sync_copy(mstage, om.at[pl.ds(rp, CH)])
            pltpu.sync_copy(idxi, oidxi.at[pl.ds(rp, CH)])

            descs = [None] * (CH // SUB)
            for sx in range(CH // SUB):
                nv = jnp.clip(cnt - (k0 + sx * SUB), 0, SUB)

                @pl.when(nv > 0)
                def _(sx=sx):
                    descs[sx] = pltpu.async_copy(
                        ptsrc_h.at[idxp.at[pl.ds(sx * SUB, SUB)]],
                        bufs[sx % 2], sems[sx % 2])

                if sx > 0:
                    drain(sx - 1, k0, base, cnt, descs)
            drain(CH // SUB - 1, k0, base, cnt, descs)

    return k(destp, cnts, coor_2d, coor_2d_o, pts_all, pts_src, zrows)


def _phase2b(idxi_all, cnts, img_t, zrows, B, N, interpret=False):
    R = B * N
    P = idxi_all.shape[0] // R
    CH = 1024
    SUB = 128
    NTILE = _NC * _NS
    per_tile = R * P // CH // NTILE
    parts = P // CH
    C = img_t.shape[1]
    mesh = plsc.VectorSubcoreMesh(
        core_axis_name="c", subcore_axis_name="s",
        num_cores=_NC, num_subcores=_NS)

    @functools.partial(
        pl.kernel,
        out_type=jax.ShapeDtypeStruct((R * P, C), jnp.float32),
        mesh=mesh,
        scratch_types=[
            pltpu.VMEM((CH,), jnp.int32),
            pltpu.VMEM((16,), jnp.int32),
            pltpu.VMEM((SUB, C), jnp.float32),
            pltpu.VMEM((SUB, C), jnp.float32),
            pltpu.VMEM((SUB, C), jnp.float32),
            pltpu.SemaphoreType.DMA,
            pltpu.SemaphoreType.DMA,
        ],
        compiler_params=pltpu.CompilerParams(needs_layout_passes=False),
        interpret=interpret,
    )
    def k(idxi_h, cnts_h, imgt_h, zrows_h, oif,
          idxi, cntb, imgbuf, imgbuf2, zerobuf, sem, sem2):
        wid = lax.axis_index("s") * _NC + lax.axis_index("c")
        iota = lax.iota(jnp.int32, 16)
        bufs = (imgbuf, imgbuf2)
        sems = (sem, sem2)

        def drain(sx, k0, base, cnt, descs):
            gbase = base + sx * SUB
            nv = jnp.clip(cnt - (k0 + sx * SUB), 0, SUB)
            buf = bufs[sx % 2]

            @pl.when(nv > 0)
            def _():
                descs[sx].wait()

                def ztail(r2, _):
                    rsp = jnp.full((16,), r2, jnp.int32)
                    for c2 in range(C // 16):
                        plsc.store_scatter(
                            buf, [rsp, c2 * 16 + iota],
                            jnp.zeros((16,), jnp.float32))
                    return 0
                lax.fori_loop(nv, SUB, ztail, 0)
                pltpu.sync_copy(buf, oif.at[pl.ds(gbase, SUB)])

            @pl.when(nv == 0)
            def _():
                pltpu.sync_copy(zerobuf, oif.at[pl.ds(gbase, SUB)])

        pltpu.sync_copy(zrows_h, zerobuf)

        for j in range(per_tile):
            chunk = wid * per_tile + j
            row = chunk // parts
            part = chunk % parts
            b = row // N
            n = row % N
            k0 = part * CH
            base = row * P + k0

            pltpu.sync_copy(idxi_h.at[pl.ds(base, CH)], idxi)
            pltpu.sync_copy(cnts_h.at[b], cntb)
            cnt = jnp.max(jnp.where(iota == n, cntb[...], 0))

            descs = [None] * (CH // SUB)
            for sx in range(CH // SUB):
                nv = jnp.clip(cnt - (k0 + sx * SUB), 0, SUB)

                @pl.when(nv > 0)
                def _(sx=sx):
                    descs[sx] = pltpu.async_copy(
                        imgt_h.at[idxi.at[pl.ds(sx * SUB, SUB)]],
                        bufs[sx % 2], sems[sx % 2])

                if sx > 0:
                    drain(sx - 1, k0, base, cnt, descs)
            drain(CH // SUB - 1, k0, base, cnt, descs)

    return k(idxi_all, cnts, img_t, zrows)


def kernel(pts_feats, coor_2d, coor_2d_o, img_feats, pts, num_points,
           interpret=False):
    B, P, C = pts_feats.shape
    N = 6
    R = B * N
    IC, H, W = img_feats.shape[1], img_feats.shape[2], img_feats.shape[3]

    img_t = jnp.transpose(img_feats, (0, 2, 3, 1)).reshape(R * H * W, IC)

    np_pad = jnp.zeros((16,), jnp.int32).at[:B].set(num_points)
    destp, cnts = _phase1(coor_2d.reshape(B, P * 3), np_pad, N,
                          interpret=interpret)

    pts_src = pts_feats.reshape(B * P, C)
    zrows = jnp.zeros((128, C), jnp.float32)
    opf, oc, oco, op, om, idxi_all = _phase2a(
        destp, cnts, coor_2d.reshape(B, P * 3), coor_2d_o.reshape(B, P * 3),
        pts.reshape(B, P * 3), pts_src, zrows, N, H * W, W,
        interpret=interpret)
    oif = _phase2b(idxi_all, cnts, img_t, zrows, B, N, interpret=interpret)

    return (
        opf.reshape(R, P, C),
        oif.reshape(R, P, IC),
        oc.reshape(R, 2, P).transpose(0, 2, 1),
        oco.reshape(R, 2, P).transpose(0, 2, 1),
        op.reshape(3, R, P).transpose(1, 2, 0),
        cnts[:, :N].reshape(R),
        om.reshape(R, P),
    )


# speedup vs baseline: 1.0111x; 1.0111x over previous
import functools

import jax
import jax.numpy as jnp
from jax import lax
from jax.experimental import pallas as pl
from jax.experimental.pallas import tpu as pltpu
from jax.experimental.pallas import tpu_sc as plsc

_NC = 2
_NS = 16
_L = 16


def _phase1(coor_2d, np_pad, N, interpret=False):
    B = coor_2d.shape[0]
    P = coor_2d.shape[1] // 3
    mesh = plsc.VectorSubcoreMesh(
        core_axis_name="c", subcore_axis_name="s",
        num_cores=_NC, num_subcores=_NS)

    @functools.partial(
        pl.kernel,
        out_type=(
            jax.ShapeDtypeStruct((B, P), jnp.int32),
            jax.ShapeDtypeStruct((B, 16), jnp.int32),
        ),
        mesh=mesh,
        scratch_types=[
            pltpu.VMEM((P * 3,), jnp.float32),
            pltpu.VMEM((P,), jnp.int32),
            pltpu.VMEM((16,), jnp.int32),
            pltpu.VMEM((16,), jnp.int32),
        ],
        compiler_params=pltpu.CompilerParams(needs_layout_passes=False),
        interpret=interpret,
    )
    def k(coor_hbm, np_hbm, destp_hbm, cnts_hbm, coorb, destb, cntrow, npb):
        wid = lax.axis_index("s") * _NC + lax.axis_index("c")

        @pl.when(wid < B)
        def _():
            b = wid
            pltpu.sync_copy(coor_hbm.at[b], coorb)
            pltpu.sync_copy(np_hbm, npb)
            iota = lax.iota(jnp.int32, 16)
            zeros16 = jnp.zeros((16,), jnp.int32)
            npv = npb[...]

            def step(v, runs):
                pidx = v * 16 + iota
                camf = plsc.load_gather(coorb, [pidx * 3])
                cam = camf.astype(jnp.int32)
                valid = pidx < jnp.max(jnp.where(iota == b, npv, 0))
                dflat = jnp.full((16,), -1, jnp.int32)
                new_runs = []
                for n in range(N):
                    msk = (cam == n) & valid
                    inc = msk.astype(jnp.int32)
                    pos = plsc.cumsum(inc) + runs[n] - 1
                    dflat = jnp.where(msk, (b * N + n) * P + pos, dflat)
                    cnt = plsc.all_reduce_population_count(msk)
                    new_runs.append(runs[n] + cnt)
                destb[pl.ds(v * 16, 16)] = dflat
                return tuple(new_runs)

            init = tuple(jnp.zeros((16,), jnp.int32) for _ in range(N))
            runs = lax.fori_loop(0, P // 16, step, init)
            total = zeros16
            for n in range(N):
                total = jnp.where(iota == n, runs[n], total)
            cntrow[...] = total
            pltpu.sync_copy(destb, destp_hbm.at[b])
            pltpu.sync_copy(cntrow, cnts_hbm.at[b])

    return k(coor_2d, np_pad)


def _gather_pipeline(src_h, idx_ref, out_h, k0, base, cnt, bufs, gsems,
                     wsems, zsem, zerobuf, iota, CH, SUB, C):
    NB = len(bufs)
    NS_ = CH // SUB
    nvs = [jnp.clip(cnt - (k0 + sx * SUB), 0, SUB) for sx in range(NS_)]
    gdescs = [None] * NS_
    wdescs = [None] * NS_
    zdescs = [None] * NS_

    def issue(sx):
        @pl.when(nvs[sx] > 0)
        def _():
            if sx >= NB:
                wdescs[sx - NB].wait()
            gdescs[sx] = pltpu.async_copy(
                src_h.at[idx_ref.at[pl.ds(sx * SUB, SUB)]],
                bufs[sx % NB], gsems[sx % NB])

    def drain(sx):
        gbase = base + sx * SUB
        nv = nvs[sx]
        buf = bufs[sx % NB]

        @pl.when(nv > 0)
        def _():
            gdescs[sx].wait()

            def ztail(r2, _):
                rsp = jnp.full((16,), r2, jnp.int32)
                for c2 in range(C // 16):
                    plsc.store_scatter(
                        buf, [rsp, c2 * 16 + iota],
                        jnp.zeros((16,), jnp.float32))
                return 0
            lax.fori_loop(nv, SUB, ztail, 0)
            wdescs[sx] = pltpu.async_copy(
                buf, out_h.at[pl.ds(gbase, SUB)], wsems[sx % NB])

        @pl.when(nv == 0)
        def _():
            zdescs[sx] = pltpu.async_copy(
                zerobuf, out_h.at[pl.ds(gbase, SUB)], zsem)

    for sx in range(NS_):
        issue(sx)
        if sx > 0:
            drain(sx - 1)
    drain(NS_ - 1)

    for sx in range(NS_):
        if sx + NB < NS_:
            pred = (nvs[sx] > 0) & (nvs[sx + NB] == 0)
        else:
            pred = nvs[sx] > 0

        @pl.when(pred)
        def _(sx=sx):
            wdescs[sx].wait()

        @pl.when(nvs[sx] == 0)
        def _(sx=sx):
            zdescs[sx].wait()


def _phase2a(destp, cnts, coor_2d, coor_2d_o, pts_all, pts_src, zrows,
             N, HW, W, interpret=False):
    B = coor_2d.shape[0]
    P = coor_2d.shape[1] // 3
    R = B * N
    CH = 1024
    SUB = 128
    NTILE = _NC * _NS
    per_tile = R * P // CH // NTILE
    parts = P // CH
    C = pts_src.shape[1]
    mesh = plsc.VectorSubcoreMesh(
        core_axis_name="c", subcore_axis_name="s",
        num_cores=_NC, num_subcores=_NS)

    @functools.partial(
        pl.kernel,
        out_type=(
            jax.ShapeDtypeStruct((R * P, C), jnp.float32),
            jax.ShapeDtypeStruct((R * 2 * P,), jnp.float32),
            jax.ShapeDtypeStruct((R * 2 * P,), jnp.float32),
            jax.ShapeDtypeStruct((3 * R * P,), jnp.float32),
            jax.ShapeDtypeStruct((R * P,), jnp.int32),
            jax.ShapeDtypeStruct((R * P,), jnp.int32),
        ),
        mesh=mesh,
        scratch_types=[
            pltpu.VMEM((P,), jnp.int32),
            pltpu.VMEM((P * 3,), jnp.float32),
            pltpu.VMEM((P * 3,), jnp.float32),
            pltpu.VMEM((P * 3,), jnp.float32),
            pltpu.VMEM((16,), jnp.int32),
            pltpu.VMEM((CH,), jnp.int32),
            pltpu.VMEM((CH,), jnp.int32),
            pltpu.VMEM((CH,), jnp.int32),
            pltpu.VMEM((SUB, C), jnp.float32),
            pltpu.VMEM((SUB, C), jnp.float32),
            pltpu.VMEM((SUB, C), jnp.float32),
            pltpu.VMEM((SUB, C), jnp.float32),
            pltpu.VMEM((CH,), jnp.float32),
            pltpu.VMEM((CH,), jnp.float32),
            pltpu.VMEM((CH,), jnp.float32),
            pltpu.VMEM((CH,), jnp.float32),
            pltpu.VMEM((CH,), jnp.float32),
            pltpu.VMEM((CH,), jnp.float32),
            pltpu.VMEM((CH,), jnp.float32),
            pltpu.VMEM((CH,), jnp.int32),
            pltpu.SemaphoreType.DMA,
            pltpu.SemaphoreType.DMA,
            pltpu.SemaphoreType.DMA,
            pltpu.SemaphoreType.DMA,
            pltpu.SemaphoreType.DMA,
            pltpu.SemaphoreType.DMA,
            pltpu.SemaphoreType.DMA,
            pltpu.SemaphoreType.DMA,
            pltpu.SemaphoreType.DMA,
        ],
        compiler_params=pltpu.CompilerParams(needs_layout_passes=False),
        interpret=interpret,
    )
    def k(destp_h, cnts_h, coor_h, cooro_h, pts_h, ptsrc_h, zrows_h,
          opf, oc, oco, op, om, oidxi,
          destb, coorb, coorob, ptsb, cntb, srcmap, idxp, idxi,
          rowbuf, rowbuf2, rowbuf3, zerobuf, cxs, cys, oxs, oys,
          pxs, pys, pzs, mstage,
          gsem0, gsem1, gsem2, wsem0, wsem1, wsem2, zsem, ssem, osem):
        wid = lax.axis_index("s") * _NC + lax.axis_index("c")
        iota = lax.iota(jnp.int32, 16)
        fz = jnp.float32(0)
        bufs = (rowbuf, rowbuf2, rowbuf3)
        gsems = (gsem0, gsem1, gsem2)
        wsems = (wsem0, wsem1, wsem2)

        pltpu.sync_copy(zrows_h, zerobuf)

        for j in range(per_tile):
            chunk = wid * per_tile + j
            row = chunk // parts
            part = chunk % parts
            b = row // N
            n = row % N
            k0 = part * CH
            base = row * P + k0

            sdescs = [
                pltpu.async_copy(destp_h.at[b], destb, ssem),
                pltpu.async_copy(coor_h.at[b], coorb, ssem),
                pltpu.async_copy(cooro_h.at[b], coorob, ssem),
                pltpu.async_copy(pts_h.at[b], ptsb, ssem),
                pltpu.async_copy(cnts_h.at[b], cntb, ssem),
            ]

            def init_map(i, _):
                srcmap[pl.ds(i * 16, 16)] = jnp.full((16,), -1, jnp.int32)
                return 0
            lax.fori_loop(0, CH // 16, init_map, 0)
            for d in sdescs:
                d.wait()

            def build_map(i, _):
                dvec = destb[pl.ds(i * 16, 16)]
                rel = dvec - base
                msk = (rel >= 0) & (rel < CH)
                plsc.store_scatter(srcmap, [rel], i * 16 + iota, mask=msk)
                return 0
            lax.fori_loop(0, P // 16, build_map, 0)

            cnt = jnp.max(jnp.where(iota == n, cntb[...], 0))

            def slots(i, _):
                pvec = srcmap[pl.ds(i * 16, 16)]
                vmsk = pvec >= 0
                psafe = jnp.where(vmsk, pvec, 0)
                p3 = psafe * 3
                cx = plsc.load_gather(coorb, [p3 + 1], mask=vmsk)
                cy = plsc.load_gather(coorb, [p3 + 2], mask=vmsk)
                ox = plsc.load_gather(coorob, [p3 + 1], mask=vmsk)
                oy = plsc.load_gather(coorob, [p3 + 2], mask=vmsk)
                px = plsc.load_gather(ptsb, [p3], mask=vmsk)
                py = plsc.load_gather(ptsb, [p3 + 1], mask=vmsk)
                pz = plsc.load_gather(ptsb, [p3 + 2], mask=vmsk)
                xi = (ox * 0.25).astype(jnp.int32)
                yi = (oy * 0.25).astype(jnp.int32)
                sl = pl.ds(i * 16, 16)
                cxs[sl] = jnp.where(vmsk, cx, fz)
                cys[sl] = jnp.where(vmsk, cy, fz)
                oxs[sl] = jnp.where(vmsk, xi.astype(jnp.float32), fz)
                oys[sl] = jnp.where(vmsk, yi.astype(jnp.float32), fz)
                pxs[sl] = jnp.where(vmsk, px, fz)
                pys[sl] = jnp.where(vmsk, py, fz)
                pzs[sl] = jnp.where(vmsk, pz, fz)
                mstage[sl] = psafe
                idxp[sl] = psafe + b * P
                idxi[sl] = jnp.where(vmsk, row * HW + yi * W + xi, 0)
                return 0
            lax.fori_loop(0, CH // 16, slots, 0)

            rk = row * 2 * P + k0
            rp = row * P + k0
            odescs = [
                pltpu.async_copy(cxs, oc.at[pl.ds(rk, CH)], osem),
                pltpu.async_copy(cys, oc.at[pl.ds(rk + P, CH)], osem),
                pltpu.async_copy(oxs, oco.at[pl.ds(rk, CH)], osem),
                pltpu.async_copy(oys, oco.at[pl.ds(rk + P, CH)], osem),
                pltpu.async_copy(pxs, op.at[pl.ds(rp, CH)], osem),
                pltpu.async_copy(pys, op.at[pl.ds(R * P + rp, CH)], osem),
                pltpu.async_copy(pzs, op.at[pl.ds(2 * R * P + rp, CH)], osem),
                pltpu.async_copy(mstage, om.at[pl.ds(rp, CH)], osem),
                pltpu.async_copy(idxi, oidxi.at[pl.ds(rp, CH)], osem),
            ]

            _gather_pipeline(ptsrc_h, idxp, opf, k0, base, cnt, bufs, gsems,
                             wsems, zsem, zerobuf, iota, CH, SUB, C)

            for d in odescs:
                d.wait()

    return k(destp, cnts, coor_2d, coor_2d_o, pts_all, pts_src, zrows)


def _phase2b(idxi_all, cnts, img_t, zrows, B, N, interpret=False):
    R = B * N
    P = idxi_all.shape[0] // R
    CH = 1024
    SUB = 128
    NTILE = _NC * _NS
    per_tile = R * P // CH // NTILE
    parts = P // CH
    C = img_t.shape[1]
    mesh = plsc.VectorSubcoreMesh(
        core_axis_name="c", subcore_axis_name="s",
        num_cores=_NC, num_subcores=_NS)

    @functools.partial(
        pl.kernel,
        out_type=jax.ShapeDtypeStruct((R * P, C), jnp.float32),
        mesh=mesh,
        scratch_types=[
            pltpu.VMEM((CH,), jnp.int32),
            pltpu.VMEM((16,), jnp.int32),
            pltpu.VMEM((SUB, C), jnp.float32),
            pltpu.VMEM((SUB, C), jnp.float32),
            pltpu.VMEM((SUB, C), jnp.float32),
            pltpu.VMEM((SUB, C), jnp.float32),
            pltpu.VMEM((SUB, C), jnp.float32),
            pltpu.SemaphoreType.DMA,
            pltpu.SemaphoreType.DMA,
            pltpu.SemaphoreType.DMA,
            pltpu.SemaphoreType.DMA,
            pltpu.SemaphoreType.DMA,
            pltpu.SemaphoreType.DMA,
            pltpu.SemaphoreType.DMA,
            pltpu.SemaphoreType.DMA,
            pltpu.SemaphoreType.DMA,
            pltpu.SemaphoreType.DMA,
        ],
        compiler_params=pltpu.CompilerParams(needs_layout_passes=False),
        interpret=interpret,
    )
    def k(idxi_h, cnts_h, imgt_h, zrows_h, oif,
          idxi, cntb, imgbuf, imgbuf2, imgbuf3, imgbuf4, zerobuf,
          gsem0, gsem1, gsem2, gsem3, wsem0, wsem1, wsem2, wsem3,
          zsem, ssem):
        wid = lax.axis_index("s") * _NC + lax.axis_index("c")
        iota = lax.iota(jnp.int32, 16)
        bufs = (imgbuf, imgbuf2, imgbuf3, imgbuf4)
        gsems = (gsem0, gsem1, gsem2, gsem3)
        wsems = (wsem0, wsem1, wsem2, wsem3)

        pltpu.sync_copy(zrows_h, zerobuf)

        for j in range(per_tile):
            chunk = wid * per_tile + j
            row = chunk // parts
            part = chunk % parts
            b = row // N
            n = row % N
            k0 = part * CH
            base = row * P + k0

            sdescs = [
                pltpu.async_copy(idxi_h.at[pl.ds(base, CH)], idxi, ssem),
                pltpu.async_copy(cnts_h.at[b], cntb, ssem),
            ]
            for d in sdescs:
                d.wait()
            cnt = jnp.max(jnp.where(iota == n, cntb[...], 0))

            _gather_pipeline(imgt_h, idxi, oif, k0, base, cnt, bufs, gsems,
                             wsems, zsem, zerobuf, iota, CH, SUB, C)

    return k(idxi_all, cnts, img_t, zrows)


def kernel(pts_feats, coor_2d, coor_2d_o, img_feats, pts, num_points,
           interpret=False):
    B, P, C = pts_feats.shape
    N = 6
    R = B * N
    IC, H, W = img_feats.shape[1], img_feats.shape[2], img_feats.shape[3]

    img_t = jnp.transpose(img_feats, (0, 2, 3, 1)).reshape(R * H * W, IC)

    np_pad = jnp.zeros((16,), jnp.int32).at[:B].set(num_points)
    destp, cnts = _phase1(coor_2d.reshape(B, P * 3), np_pad, N,
                          interpret=interpret)

    pts_src = pts_feats.reshape(B * P, C)
    zrows = jnp.zeros((128, C), jnp.float32)
    opf, oc, oco, op, om, idxi_all = _phase2a(
        destp, cnts, coor_2d.reshape(B, P * 3), coor_2d_o.reshape(B, P * 3),
        pts.reshape(B, P * 3), pts_src, zrows, N, H * W, W,
        interpret=interpret)
    oif = _phase2b(idxi_all, cnts, img_t, zrows, B, N, interpret=interpret)

    return (
        opf.reshape(R, P, C),
        oif.reshape(R, P, IC),
        oc.reshape(R, 2, P).transpose(0, 2, 1),
        oco.reshape(R, 2, P).transpose(0, 2, 1),
        op.reshape(3, R, P).transpose(1, 2, 0),
        cnts[:, :N].reshape(R),
        om.reshape(R, P),
    )


# speedup vs baseline: 1.0615x; 1.0498x over previous
import functools

import jax
import jax.numpy as jnp
from jax import lax
from jax.experimental import pallas as pl
from jax.experimental.pallas import tpu as pltpu
from jax.experimental.pallas import tpu_sc as plsc

_NC = 2
_NS = 16
_L = 16


def _phase1(coor_2d, np_pad, N, interpret=False):
    B = coor_2d.shape[0]
    P = coor_2d.shape[1] // 3
    mesh = plsc.VectorSubcoreMesh(
        core_axis_name="c", subcore_axis_name="s",
        num_cores=_NC, num_subcores=_NS)

    @functools.partial(
        pl.kernel,
        out_type=(
            jax.ShapeDtypeStruct((B, P), jnp.int32),
            jax.ShapeDtypeStruct((B, 16), jnp.int32),
        ),
        mesh=mesh,
        scratch_types=[
            pltpu.VMEM((P * 3,), jnp.float32),
            pltpu.VMEM((P,), jnp.int32),
            pltpu.VMEM((16,), jnp.int32),
            pltpu.VMEM((16,), jnp.int32),
        ],
        compiler_params=pltpu.CompilerParams(needs_layout_passes=False),
        interpret=interpret,
    )
    def k(coor_hbm, np_hbm, destp_hbm, cnts_hbm, coorb, destb, cntrow, npb):
        wid = lax.axis_index("s") * _NC + lax.axis_index("c")

        @pl.when(wid < B)
        def _():
            b = wid
            pltpu.sync_copy(coor_hbm.at[b], coorb)
            pltpu.sync_copy(np_hbm, npb)
            iota = lax.iota(jnp.int32, 16)
            zeros16 = jnp.zeros((16,), jnp.int32)
            npv = npb[...]

            def step(v, runs):
                pidx = v * 16 + iota
                camf = plsc.load_gather(coorb, [pidx * 3])
                cam = camf.astype(jnp.int32)
                valid = pidx < jnp.max(jnp.where(iota == b, npv, 0))
                dflat = jnp.full((16,), -1, jnp.int32)
                new_runs = []
                for n in range(N):
                    msk = (cam == n) & valid
                    inc = msk.astype(jnp.int32)
                    pos = plsc.cumsum(inc) + runs[n] - 1
                    dflat = jnp.where(msk, (b * N + n) * P + pos, dflat)
                    cnt = plsc.all_reduce_population_count(msk)
                    new_runs.append(runs[n] + cnt)
                destb[pl.ds(v * 16, 16)] = dflat
                return tuple(new_runs)

            init = tuple(jnp.zeros((16,), jnp.int32) for _ in range(N))
            runs = lax.fori_loop(0, P // 16, step, init)
            total = zeros16
            for n in range(N):
                total = jnp.where(iota == n, runs[n], total)
            cntrow[...] = total
            pltpu.sync_copy(destb, destp_hbm.at[b])
            pltpu.sync_copy(cntrow, cnts_hbm.at[b])

    return k(coor_2d, np_pad)


def _phase2(destp, cnts, cxy, oxy, pts_all, pts_src, img_t, zrows,
            N, HW, W, interpret=False):
    B = cxy.shape[0]
    P = cxy.shape[1] // 2
    R = B * N
    CH = 1024
    SUB = 128
    NB = 2
    NTILE = _NC * _NS
    per_tile = R * P // CH // NTILE
    parts = P // CH
    C = pts_src.shape[1]
    mesh = plsc.VectorSubcoreMesh(
        core_axis_name="c", subcore_axis_name="s",
        num_cores=_NC, num_subcores=_NS)

    @functools.partial(
        pl.kernel,
        out_type=(
            jax.ShapeDtypeStruct((R * P, C), jnp.float32),
            jax.ShapeDtypeStruct((R * P, C), jnp.float32),
            jax.ShapeDtypeStruct((R * 2 * P,), jnp.float32),
            jax.ShapeDtypeStruct((R * 2 * P,), jnp.float32),
            jax.ShapeDtypeStruct((3 * R * P,), jnp.float32),
            jax.ShapeDtypeStruct((R * P,), jnp.int32),
        ),
        mesh=mesh,
        scratch_types=[
            pltpu.VMEM((P,), jnp.int32),
            pltpu.VMEM((P * 2,), jnp.float32),
            pltpu.VMEM((P * 2,), jnp.float32),
            pltpu.VMEM((P * 3,), jnp.float32),
            pltpu.VMEM((16,), jnp.int32),
            pltpu.VMEM((CH,), jnp.int32),
            pltpu.VMEM((CH,), jnp.int32),
            pltpu.VMEM((CH,), jnp.int32),
            pltpu.VMEM((SUB, C), jnp.float32),
            pltpu.VMEM((SUB, C), jnp.float32),
            pltpu.VMEM((SUB, C), jnp.float32),
            pltpu.VMEM((SUB, C), jnp.float32),
            pltpu.VMEM((SUB, C), jnp.float32),
            pltpu.VMEM((CH,), jnp.float32),
            pltpu.VMEM((CH,), jnp.float32),
            pltpu.VMEM((CH,), jnp.float32),
            pltpu.VMEM((CH,), jnp.float32),
            pltpu.VMEM((CH,), jnp.float32),
            pltpu.VMEM((CH,), jnp.float32),
            pltpu.VMEM((CH,), jnp.float32),
            pltpu.VMEM((CH,), jnp.int32),
            pltpu.SemaphoreType.DMA,
            pltpu.SemaphoreType.DMA,
            pltpu.SemaphoreType.DMA,
            pltpu.SemaphoreType.DMA,
            pltpu.SemaphoreType.DMA,
            pltpu.SemaphoreType.DMA,
            pltpu.SemaphoreType.DMA,
            pltpu.SemaphoreType.DMA,
            pltpu.SemaphoreType.DMA,
            pltpu.SemaphoreType.DMA,
            pltpu.SemaphoreType.DMA,
        ],
        compiler_params=pltpu.CompilerParams(needs_layout_passes=False),
        interpret=interpret,
    )
    def k(destp_h, cnts_h, cxy_h, oxy_h, pts_h, ptsrc_h, imgt_h, zrows_h,
          opf, oif, oc, oco, op, om,
          destb, cxyb, oxyb, ptsb, cntb, srcmap, idxp, idxi,
          pbuf0, pbuf1, ibuf0, ibuf1, zerobuf,
          cxs, cys, oxs, oys, pxs, pys, pzs, mstage,
          pgsem0, pgsem1, igsem0, igsem1,
          pwsem0, pwsem1, iwsem0, iwsem1, zsem, ssem, osem):
        wid = lax.axis_index("s") * _NC + lax.axis_index("c")
        iota = lax.iota(jnp.int32, 16)
        fz = jnp.float32(0)
        pbufs = (pbuf0, pbuf1)
        ibufs = (ibuf0, ibuf1)
        pgsems = (pgsem0, pgsem1)
        igsems = (igsem0, igsem1)
        pwsems = (pwsem0, pwsem1)
        iwsems = (iwsem0, iwsem1)

        pltpu.sync_copy(zrows_h, zerobuf)

        for j in range(per_tile):
            chunk = wid * per_tile + j
            row = chunk // parts
            part = chunk % parts
            b = row // N
            n = row % N
            k0 = part * CH
            base = row * P + k0

            sdescs = [
                pltpu.async_copy(destp_h.at[b], destb, ssem),
                pltpu.async_copy(cxy_h.at[b], cxyb, ssem),
                pltpu.async_copy(oxy_h.at[b], oxyb, ssem),
                pltpu.async_copy(pts_h.at[b], ptsb, ssem),
                pltpu.async_copy(cnts_h.at[b], cntb, ssem),
            ]

            def init_map(i, _):
                srcmap[pl.ds(i * 16, 16)] = jnp.full((16,), -1, jnp.int32)
                return 0
            lax.fori_loop(0, CH // 16, init_map, 0)
            for d in sdescs:
                d.wait()

            def build_map(i, _):
                dvec = destb[pl.ds(i * 16, 16)]
                rel = dvec - base
                msk = (rel >= 0) & (rel < CH)
                plsc.store_scatter(srcmap, [rel], i * 16 + iota, mask=msk)
                return 0
            lax.fori_loop(0, P // 16, build_map, 0)

            cnt = jnp.max(jnp.where(iota == n, cntb[...], 0))

            def slots(i, _):
                pvec = srcmap[pl.ds(i * 16, 16)]
                vmsk = pvec >= 0
                psafe = jnp.where(vmsk, pvec, 0)
                p2 = psafe * 2
                p3 = psafe * 3
                cx = plsc.load_gather(cxyb, [p2], mask=vmsk)
                cy = plsc.load_gather(cxyb, [p2 + 1], mask=vmsk)
                ox = plsc.load_gather(oxyb, [p2], mask=vmsk)
                oy = plsc.load_gather(oxyb, [p2 + 1], mask=vmsk)
                px = plsc.load_gather(ptsb, [p3], mask=vmsk)
                py = plsc.load_gather(ptsb, [p3 + 1], mask=vmsk)
                pz = plsc.load_gather(ptsb, [p3 + 2], mask=vmsk)
                xi = (ox * 0.25).astype(jnp.int32)
                yi = (oy * 0.25).astype(jnp.int32)
                sl = pl.ds(i * 16, 16)
                cxs[sl] = jnp.where(vmsk, cx, fz)
                cys[sl] = jnp.where(vmsk, cy, fz)
                oxs[sl] = jnp.where(vmsk, xi.astype(jnp.float32), fz)
                oys[sl] = jnp.where(vmsk, yi.astype(jnp.float32), fz)
                pxs[sl] = jnp.where(vmsk, px, fz)
                pys[sl] = jnp.where(vmsk, py, fz)
                pzs[sl] = jnp.where(vmsk, pz, fz)
                mstage[sl] = psafe
                idxp[sl] = psafe + b * P
                idxi[sl] = jnp.where(vmsk, row * HW + yi * W + xi, 0)
                return 0
            lax.fori_loop(0, CH // 16, slots, 0)

            rk = row * 2 * P + k0
            rp = row * P + k0
            odescs = [
                pltpu.async_copy(cxs, oc.at[pl.ds(rk, CH)], osem),
                pltpu.async_copy(cys, oc.at[pl.ds(rk + P, CH)], osem),
                pltpu.async_copy(oxs, oco.at[pl.ds(rk, CH)], osem),
                pltpu.async_copy(oys, oco.at[pl.ds(rk + P, CH)], osem),
                pltpu.async_copy(pxs, op.at[pl.ds(rp, CH)], osem),
                pltpu.async_copy(pys, op.at[pl.ds(R * P + rp, CH)], osem),
                pltpu.async_copy(pzs, op.at[pl.ds(2 * R * P + rp, CH)], osem),
                pltpu.async_copy(mstage, om.at[pl.ds(rp, CH)], osem),
            ]

            NS_ = CH // SUB
            nvs = [jnp.clip(cnt - (k0 + sx * SUB), 0, SUB)
                   for sx in range(NS_)]
            pg = [None] * NS_
            ig = [None] * NS_
            pw = [None] * NS_
            iw = [None] * NS_
            zd = [None] * NS_

            def issue(sx):
                @pl.when(nvs[sx] > 0)
                def _():
                    if sx >= NB:
                        pw[sx - NB].wait()
                        iw[sx - NB].wait()
                    pg[sx] = pltpu.async_copy(
                        ptsrc_h.at[idxp.at[pl.ds(sx * SUB, SUB)]],
                        pbufs[sx % NB], pgsems[sx % NB])
                    ig[sx] = pltpu.async_copy(
                        imgt_h.at[idxi.at[pl.ds(sx * SUB, SUB)]],
                        ibufs[sx % NB], igsems[sx % NB])

            def drain(sx):
                gbase = base + sx * SUB
                nv = nvs[sx]
                pb = pbufs[sx % NB]
                ib = ibufs[sx % NB]

                @pl.when(nv > 0)
                def _():
                    pg[sx].wait()
                    ig[sx].wait()

                    def ztail(r2, _):
                        rsp = jnp.full((16,), r2, jnp.int32)
                        for c2 in range(C // 16):
                            plsc.store_scatter(
                                pb, [rsp, c2 * 16 + iota],
                                jnp.zeros((16,), jnp.float32))
                            plsc.store_scatter(
                                ib, [rsp, c2 * 16 + iota],
                                jnp.zeros((16,), jnp.float32))
                        return 0
                    lax.fori_loop(nv, SUB, ztail, 0)
                    pw[sx] = pltpu.async_copy(
                        pb, opf.at[pl.ds(gbase, SUB)], pwsems[sx % NB])
                    iw[sx] = pltpu.async_copy(
                        ib, oif.at[pl.ds(gbase, SUB)], iwsems[sx % NB])

                @pl.when(nv == 0)
                def _():
                    zd[sx] = (
                        pltpu.async_copy(
                            zerobuf, opf.at[pl.ds(gbase, SUB)], zsem),
                        pltpu.async_copy(
                            zerobuf, oif.at[pl.ds(gbase, SUB)], zsem),
                    )

            for sx in range(NS_):
                issue(sx)
                if sx > 0:
                    drain(sx - 1)
            drain(NS_ - 1)

            for sx in range(NS_):
                if sx + NB < NS_:
                    pred = (nvs[sx] > 0) & (nvs[sx + NB] == 0)
                else:
                    pred = nvs[sx] > 0

                @pl.when(pred)
                def _(sx=sx):
                    pw[sx].wait()
                    iw[sx].wait()

                @pl.when(nvs[sx] == 0)
                def _(sx=sx):
                    zd[sx][0].wait()
                    zd[sx][1].wait()

            for d in odescs:
                d.wait()

    return k(destp, cnts, cxy, oxy, pts_all, pts_src, img_t, zrows)


def kernel(pts_feats, coor_2d, coor_2d_o, img_feats, pts, num_points,
           interpret=False):
    B, P, C = pts_feats.shape
    N = 6
    R = B * N
    IC, H, W = img_feats.shape[1], img_feats.shape[2], img_feats.shape[3]

    img_t = jnp.transpose(img_feats, (0, 2, 3, 1)).reshape(R * H * W, IC)

    np_pad = jnp.zeros((16,), jnp.int32).at[:B].set(num_points)
    destp, cnts = _phase1(coor_2d.reshape(B, P * 3), np_pad, N,
                          interpret=interpret)

    pts_src = pts_feats.reshape(B * P, C)
    zrows = jnp.zeros((128, C), jnp.float32)
    cxy = coor_2d[:, :, 1:3].reshape(B, P * 2)
    oxy = coor_2d_o[:, :, 1:3].reshape(B, P * 2)
    opf, oif, oc, oco, op, om = _phase2(
        destp, cnts, cxy, oxy, pts.reshape(B, P * 3), pts_src, img_t, zrows,
        N, H * W, W, interpret=interpret)

    return (
        opf.reshape(R, P, C),
        oif.reshape(R, P, IC),
        oc.reshape(R, 2, P).transpose(0, 2, 1),
        oco.reshape(R, 2, P).transpose(0, 2, 1),
        op.reshape(3, R, P).transpose(1, 2, 0),
        cnts[:, :N].reshape(R),
        om.reshape(R, P),
    )


# speedup vs baseline: 1.0700x; 1.0081x over previous
import functools

import jax
import jax.numpy as jnp
from jax import lax
from jax.experimental import pallas as pl
from jax.experimental.pallas import tpu as pltpu
from jax.experimental.pallas import tpu_sc as plsc

_NC = 2
_NS = 16
_L = 16


def _phase1(coor_2d, np_pad, N):
    B = coor_2d.shape[0]
    P = coor_2d.shape[1] // 3
    mesh = plsc.VectorSubcoreMesh(
        core_axis_name="c", subcore_axis_name="s",
        num_cores=_NC, num_subcores=_NS)

    @functools.partial(
        pl.kernel,
        out_type=(
            jax.ShapeDtypeStruct((B, P), jnp.int32),
            jax.ShapeDtypeStruct((B, 16), jnp.int32),
        ),
        mesh=mesh,
        scratch_types=[
            pltpu.VMEM((P * 3,), jnp.float32),
            pltpu.VMEM((P,), jnp.int32),
            pltpu.VMEM((16,), jnp.int32),
            pltpu.VMEM((16,), jnp.int32),
        ],
        compiler_params=pltpu.CompilerParams(needs_layout_passes=False),
    )
    def k(coor_hbm, np_hbm, destp_hbm, cnts_hbm, coorb, destb, cntrow, npb):
        wid = lax.axis_index("s") * _NC + lax.axis_index("c")

        @pl.when(wid < B)
        def _():
            b = wid
            pltpu.sync_copy(coor_hbm.at[b], coorb)
            pltpu.sync_copy(np_hbm, npb)
            iota = lax.iota(jnp.int32, 16)
            zeros16 = jnp.zeros((16,), jnp.int32)
            npv = npb[...]

            def step(v, runs):
                pidx = v * 16 + iota
                camf = plsc.load_gather(coorb, [pidx * 3])
                cam = camf.astype(jnp.int32)
                valid = pidx < jnp.max(jnp.where(iota == b, npv, 0))
                dflat = jnp.full((16,), -1, jnp.int32)
                new_runs = []
                for n in range(N):
                    msk = (cam == n) & valid
                    inc = msk.astype(jnp.int32)
                    pos = plsc.cumsum(inc) + runs[n] - 1
                    dflat = jnp.where(msk, (b * N + n) * P + pos, dflat)
                    cnt = plsc.all_reduce_population_count(msk)
                    new_runs.append(runs[n] + cnt)
                destb[pl.ds(v * 16, 16)] = dflat
                return tuple(new_runs)

            init = tuple(jnp.zeros((16,), jnp.int32) for _ in range(N))
            runs = lax.fori_loop(0, P // 16, step, init)
            total = zeros16
            for n in range(N):
                total = jnp.where(iota == n, runs[n], total)
            cntrow[...] = total
            pltpu.sync_copy(destb, destp_hbm.at[b])
            pltpu.sync_copy(cntrow, cnts_hbm.at[b])

    return k(coor_2d, np_pad)


def _phase2(destp, cnts, cxy, oxy, pts_all, pts_src, img_t, zrows,
            N, HW, W):
    B = cxy.shape[0]
    P = cxy.shape[1] // 2
    R = B * N
    CH = 1024
    SUB = 128
    NB = 2
    NTILE = _NC * _NS
    per_tile = R * P // CH // NTILE
    parts = P // CH
    C = pts_src.shape[1]
    mesh = plsc.VectorSubcoreMesh(
        core_axis_name="c", subcore_axis_name="s",
        num_cores=_NC, num_subcores=_NS)

    @functools.partial(
        pl.kernel,
        out_type=(
            jax.ShapeDtypeStruct((R * P, C), jnp.float32),
            jax.ShapeDtypeStruct((R * P, C), jnp.float32),
            jax.ShapeDtypeStruct((R * 2 * P,), jnp.float32),
            jax.ShapeDtypeStruct((R * 2 * P,), jnp.float32),
            jax.ShapeDtypeStruct((3 * R * P,), jnp.float32),
            jax.ShapeDtypeStruct((R * P,), jnp.int32),
        ),
        mesh=mesh,
        scratch_types=[
            pltpu.VMEM((P,), jnp.int32),
            pltpu.VMEM((P * 2,), jnp.float32),
            pltpu.VMEM((P * 2,), jnp.float32),
            pltpu.VMEM((P * 3,), jnp.float32),
            pltpu.VMEM((16,), jnp.int32),
            pltpu.VMEM((CH,), jnp.int32),
            pltpu.VMEM((CH,), jnp.int32),
            pltpu.VMEM((CH,), jnp.int32),
            pltpu.VMEM((SUB, C), jnp.float32),
            pltpu.VMEM((SUB, C), jnp.float32),
            pltpu.VMEM((SUB, C), jnp.float32),
            pltpu.VMEM((SUB, C), jnp.float32),
            pltpu.VMEM((SUB, C), jnp.float32),
            pltpu.VMEM((CH,), jnp.float32),
            pltpu.VMEM((CH,), jnp.float32),
            pltpu.VMEM((CH,), jnp.float32),
            pltpu.VMEM((CH,), jnp.float32),
            pltpu.VMEM((CH,), jnp.float32),
            pltpu.VMEM((CH,), jnp.float32),
            pltpu.VMEM((CH,), jnp.float32),
            pltpu.VMEM((CH,), jnp.int32),
            pltpu.SemaphoreType.DMA,
            pltpu.SemaphoreType.DMA,
            pltpu.SemaphoreType.DMA,
            pltpu.SemaphoreType.DMA,
            pltpu.SemaphoreType.DMA,
            pltpu.SemaphoreType.DMA,
            pltpu.SemaphoreType.DMA,
            pltpu.SemaphoreType.DMA,
            pltpu.SemaphoreType.DMA,
            pltpu.SemaphoreType.DMA,
            pltpu.SemaphoreType.DMA,
        ],
        compiler_params=pltpu.CompilerParams(needs_layout_passes=False),
    )
    def k(destp_h, cnts_h, cxy_h, oxy_h, pts_h, ptsrc_h, imgt_h, zrows_h,
          opf, oif, oc, oco, op, om,
          destb, cxyb, oxyb, ptsb, cntb, srcmap, idxp, idxi,
          pbuf0, pbuf1, ibuf0, ibuf1, zerobuf,
          cxs, cys, oxs, oys, pxs, pys, pzs, mstage,
          pgsem0, pgsem1, igsem0, igsem1,
          pwsem0, pwsem1, iwsem0, iwsem1, zsem, ssem, osem):
        wid = lax.axis_index("s") * _NC + lax.axis_index("c")
        iota = lax.iota(jnp.int32, 16)
        fz = jnp.float32(0)
        pbufs = (pbuf0, pbuf1)
        ibufs = (ibuf0, ibuf1)
        pgsems = (pgsem0, pgsem1)
        igsems = (igsem0, igsem1)
        pwsems = (pwsem0, pwsem1)
        iwsems = (iwsem0, iwsem1)

        pltpu.sync_copy(zrows_h, zerobuf)

        for j in range(per_tile):
            chunk = wid * per_tile + j
            row = chunk // parts
            part = chunk % parts
            b = row // N
            n = row % N
            k0 = part * CH
            base = row * P + k0

            sdescs = [
                pltpu.async_copy(destp_h.at[b], destb, ssem),
                pltpu.async_copy(cxy_h.at[b], cxyb, ssem),
                pltpu.async_copy(oxy_h.at[b], oxyb, ssem),
                pltpu.async_copy(pts_h.at[b], ptsb, ssem),
                pltpu.async_copy(cnts_h.at[b], cntb, ssem),
            ]

            def init_map(i, _):
                srcmap[pl.ds(i * 16, 16)] = jnp.full((16,), -1, jnp.int32)
                return 0
            lax.fori_loop(0, CH // 16, init_map, 0)
            for d in sdescs:
                d.wait()

            def build_map(i, _):
                dvec = destb[pl.ds(i * 16, 16)]
                rel = dvec - base
                msk = (rel >= 0) & (rel < CH)
                plsc.store_scatter(srcmap, [rel], i * 16 + iota, mask=msk)
                return 0
            lax.fori_loop(0, P // 16, build_map, 0)

            cnt = jnp.max(jnp.where(iota == n, cntb[...], 0))

            def slots(i, _):
                pvec = srcmap[pl.ds(i * 16, 16)]
                vmsk = pvec >= 0
                psafe = jnp.where(vmsk, pvec, 0)
                p2 = psafe * 2
                p3 = psafe * 3
                cx = plsc.load_gather(cxyb, [p2], mask=vmsk)
                cy = plsc.load_gather(cxyb, [p2 + 1], mask=vmsk)
                ox = plsc.load_gather(oxyb, [p2], mask=vmsk)
                oy = plsc.load_gather(oxyb, [p2 + 1], mask=vmsk)
                px = plsc.load_gather(ptsb, [p3], mask=vmsk)
                py = plsc.load_gather(ptsb, [p3 + 1], mask=vmsk)
                pz = plsc.load_gather(ptsb, [p3 + 2], mask=vmsk)
                xi = (ox * 0.25).astype(jnp.int32)
                yi = (oy * 0.25).astype(jnp.int32)
                sl = pl.ds(i * 16, 16)
                cxs[sl] = jnp.where(vmsk, cx, fz)
                cys[sl] = jnp.where(vmsk, cy, fz)
                oxs[sl] = jnp.where(vmsk, xi.astype(jnp.float32), fz)
                oys[sl] = jnp.where(vmsk, yi.astype(jnp.float32), fz)
                pxs[sl] = jnp.where(vmsk, px, fz)
                pys[sl] = jnp.where(vmsk, py, fz)
                pzs[sl] = jnp.where(vmsk, pz, fz)
                mstage[sl] = psafe
                idxp[sl] = psafe + b * P
                idxi[sl] = jnp.where(vmsk, row * HW + yi * W + xi, 0)
                return 0
            lax.fori_loop(0, CH // 16, slots, 0)

            rk = row * 2 * P + k0
            rp = row * P + k0
            odescs = [
                pltpu.async_copy(cxs, oc.at[pl.ds(rk, CH)], osem),
                pltpu.async_copy(cys, oc.at[pl.ds(rk + P, CH)], osem),
                pltpu.async_copy(oxs, oco.at[pl.ds(rk, CH)], osem),
                pltpu.async_copy(oys, oco.at[pl.ds(rk + P, CH)], osem),
                pltpu.async_copy(pxs, op.at[pl.ds(rp, CH)], osem),
                pltpu.async_copy(pys, op.at[pl.ds(R * P + rp, CH)], osem),
                pltpu.async_copy(pzs, op.at[pl.ds(2 * R * P + rp, CH)], osem),
                pltpu.async_copy(mstage, om.at[pl.ds(rp, CH)], osem),
            ]

            NS_ = CH // SUB
            nvs = [jnp.clip(cnt - (k0 + sx * SUB), 0, SUB)
                   for sx in range(NS_)]
            pg = [None] * NS_
            ig = [None] * NS_
            pw = [None] * NS_
            iw = [None] * NS_
            zd = [None] * NS_

            def issue(sx):
                @pl.when(nvs[sx] > 0)
                def _():
                    if sx >= NB:
                        pw[sx - NB].wait()
                        iw[sx - NB].wait()
                    pg[sx] = pltpu.async_copy(
                        ptsrc_h.at[idxp.at[pl.ds(sx * SUB, SUB)]],
                        pbufs[sx % NB], pgsems[sx % NB])
                    ig[sx] = pltpu.async_copy(
                        imgt_h.at[idxi.at[pl.ds(sx * SUB, SUB)]],
                        ibufs[sx % NB], igsems[sx % NB])

            def drain(sx):
                gbase = base + sx * SUB
                nv = nvs[sx]
                pb = pbufs[sx % NB]
                ib = ibufs[sx % NB]

                @pl.when(nv > 0)
                def _():
                    pg[sx].wait()
                    ig[sx].wait()

                    def ztail(r2, _):
                        rsp = jnp.full((16,), r2, jnp.int32)
                        for c2 in range(C // 16):
                            plsc.store_scatter(
                                pb, [rsp, c2 * 16 + iota],
                                jnp.zeros((16,), jnp.float32))
                            plsc.store_scatter(
                                ib, [rsp, c2 * 16 + iota],
                                jnp.zeros((16,), jnp.float32))
                        return 0
                    lax.fori_loop(nv, SUB, ztail, 0)
                    pw[sx] = pltpu.async_copy(
                        pb, opf.at[pl.ds(gbase, SUB)], pwsems[sx % NB])
                    iw[sx] = pltpu.async_copy(
                        ib, oif.at[pl.ds(gbase, SUB)], iwsems[sx % NB])

                @pl.when(nv == 0)
                def _():
                    zd[sx] = (
                        pltpu.async_copy(
                            zerobuf, opf.at[pl.ds(gbase, SUB)], zsem),
                        pltpu.async_copy(
                            zerobuf, oif.at[pl.ds(gbase, SUB)], zsem),
                    )

            for sx in range(NS_):
                issue(sx)
                if sx > 0:
                    drain(sx - 1)
            drain(NS_ - 1)

            for sx in range(NS_):
                if sx + NB < NS_:
                    pred = (nvs[sx] > 0) & (nvs[sx + NB] == 0)
                else:
                    pred = nvs[sx] > 0

                @pl.when(pred)
                def _(sx=sx):
                    pw[sx].wait()
                    iw[sx].wait()

                @pl.when(nvs[sx] == 0)
                def _(sx=sx):
                    zd[sx][0].wait()
                    zd[sx][1].wait()

            for d in odescs:
                d.wait()

    return k(destp, cnts, cxy, oxy, pts_all, pts_src, img_t, zrows)


def kernel(pts_feats, coor_2d, coor_2d_o, img_feats, pts, num_points):
    B, P, C = pts_feats.shape
    N = 6
    R = B * N
    IC, H, W = img_feats.shape[1], img_feats.shape[2], img_feats.shape[3]

    img_t = jnp.transpose(img_feats, (0, 2, 3, 1)).reshape(R * H * W, IC)

    np_pad = jnp.zeros((16,), jnp.int32).at[:B].set(num_points)
    destp, cnts = _phase1(coor_2d.reshape(B, P * 3), np_pad, N)

    pts_src = pts_feats.reshape(B * P, C)
    zrows = jnp.zeros((128, C), jnp.float32)
    cxy = coor_2d[:, :, 1:3].reshape(B, P * 2)
    oxy = coor_2d_o[:, :, 1:3].reshape(B, P * 2)
    opf, oif, oc, oco, op, om = _phase2(
        destp, cnts, cxy, oxy, pts.reshape(B, P * 3), pts_src, img_t, zrows,
        N, H * W, W)

    return (
        opf.reshape(R, P, C),
        oif.reshape(R, P, IC),
        oc.reshape(R, 2, P).transpose(0, 2, 1),
        oco.reshape(R, 2, P).transpose(0, 2, 1),
        op.reshape(3, R, P).transpose(1, 2, 0),
        cnts[:, :N].reshape(R),
        om.reshape(R, P),
    )
